# baseline (speedup 1.0000x reference)
import functools
import math

import jax
import jax.numpy as jnp
from jax import lax
from jax.experimental import pallas as pl
from jax.experimental.pallas import tpu as pltpu
from jax.experimental.pallas import tpu_sc as plsc

B, L, D, H, FACTOR = 4, 4096, 2048, 16, 5
Dh = D // H
U = max(1, min(FACTOR * int(math.ceil(math.log(L + 1))), L))
SCALE = 1.0 / math.sqrt(Dh)
UP = 48
NP = H * UP
NG = B * NP
NW = 32
LBLK = 512
KBLK = 512
NKB = L // KBLK
NEG = float("-inf")


def _p1a_body(ksamp_ref, wk_ref, bk_ref, ks_ref):
    ks = lax.dot_general(ksamp_ref[0].astype(jnp.bfloat16),
                         wk_ref[...].astype(jnp.bfloat16),
                         (((1,), (1,)), ((), ())),
                         preferred_element_type=jnp.float32)
    ks_ref[0, 0] = (ks + bk_ref[...][None, :]).astype(jnp.bfloat16)


def _p1a(ksamp, Wk, bk):
    return pl.pallas_call(
        _p1a_body,
        grid=(B, H),
        in_specs=[
            pl.BlockSpec((1, U, D), lambda b, h: (b, 0, 0)),
            pl.BlockSpec((Dh, D), lambda b, h: (h, 0)),
            pl.BlockSpec((Dh,), lambda b, h: (h,)),
        ],
        out_specs=pl.BlockSpec((1, 1, U, Dh), lambda b, h: (b, h, 0, 0)),
        out_shape=jax.ShapeDtypeStruct((B, H, U, Dh), jnp.bfloat16),
    )(ksamp, Wk, bk)


def _p1b_body(q_ref, wq_ref, ks_ref, bq_ref, m_ref):
    qb = lax.dot_general(q_ref[0].astype(jnp.bfloat16), wq_ref[...],
                         (((1,), (1,)), ((), ())),
                         preferred_element_type=jnp.float32)
    qbb = (qb + bq_ref[...][None, :]).astype(jnp.bfloat16)
    li = pl.program_id(1)
    nr = LBLK // 128
    for h in range(H):
        qh = qbb[:, h * Dh:(h + 1) * Dh]
        s = lax.dot_general(qh, ks_ref[0, h], (((1,), (1,)), ((), ())),
                            preferred_element_type=jnp.float32) * SCALE
        m = jnp.max(s, axis=1) - jnp.sum(s, axis=1) / jnp.float32(U)
        m_ref[0, h, pl.ds(li * nr, nr), :] = m.reshape(nr, 128)


def _p1b(q, Wqb, Ks, bq):
    return pl.pallas_call(
        _p1b_body,
        grid=(B, L // LBLK),
        in_specs=[
            pl.BlockSpec((1, LBLK, D), lambda b, l: (b, l, 0)),
            pl.BlockSpec((D, D), lambda b, l: (0, 0)),
            pl.BlockSpec((1, H, U, Dh), lambda b, l: (b, 0, 0, 0)),
            pl.BlockSpec((D,), lambda b, l: (0,)),
        ],
        out_specs=pl.BlockSpec((1, H, L // 128, 128),
                               lambda b, l: (b, 0, 0, 0)),
        out_shape=jax.ShapeDtypeStruct((B, H, L // 128, 128), jnp.float32),
    )(q, Wqb, Ks, bq)


def _p2_body(m_ref, ti_ref):
    x = m_ref[0, 0]
    lin = (lax.broadcasted_iota(jnp.int32, (L // 128, 128), 0) * 128
           + lax.broadcasted_iota(jnp.int32, (L // 128, 128), 1))
    lane = lax.broadcasted_iota(jnp.int32, (1, 64), 1)
    sel = jnp.zeros((1, 64), jnp.int32)
    for i in range(U):
        mx = jnp.max(x)
        idx = jnp.min(jnp.where(x == mx, lin, L))
        sel = jnp.where(lane == i, idx, sel)
        x = jnp.where(lin == idx, NEG, x)
    ti_ref[0, 0] = sel


def _p2(M):
    return pl.pallas_call(
        _p2_body,
        grid=(B, H),
        in_specs=[pl.BlockSpec((1, 1, L // 128, 128), lambda b, h: (b, h, 0, 0))],
        out_specs=pl.BlockSpec((1, 1, 1, 64), lambda b, h: (b, h, 0, 0)),
        out_shape=jax.ShapeDtypeStruct((B, H, 1, 64), jnp.int32),
    )(M)


def _sc_gather(nrows, chunk):
    per_w = nrows // NW
    nch = per_w // chunk
    mesh = plsc.VectorSubcoreMesh(core_axis_name="c", subcore_axis_name="s")

    @functools.partial(
        pl.kernel, mesh=mesh,
        out_type=jax.ShapeDtypeStruct((nrows, D), jnp.float32),
        scratch_types=[
            pltpu.VMEM((chunk,), jnp.int32),
            pltpu.VMEM((chunk, D), jnp.float32),
            pltpu.SemaphoreType.DMA,
        ],
    )
    def g(table_hbm, idx_hbm, out_hbm, idx_v, rows_v, sem):
        wid = lax.axis_index("s") * 2 + lax.axis_index("c")
        for j in range(nch):
            pltpu.sync_copy(idx_hbm.at[wid, j], idx_v)
            pltpu.async_copy(table_hbm.at[idx_v], rows_v, sem).wait()
            pltpu.sync_copy(rows_v, out_hbm.at[pl.ds(wid * per_w + j * chunk,
                                                     chunk)])
    return g


def _sc_scatter():
    per_w = NG // NW
    chunk = 16
    nch = per_w // chunk
    mesh = plsc.VectorSubcoreMesh(core_axis_name="c", subcore_axis_name="s")

    @functools.partial(
        pl.kernel, mesh=mesh,
        out_type=(),
        scratch_types=[
            pltpu.VMEM((chunk,), jnp.int32),
            pltpu.VMEM((chunk, D), jnp.float32),
            pltpu.SemaphoreType.DMA,
        ],
    )
    def s(r_hbm, idx_hbm, out_ref, idx_v, rows_v, sem):
        wid = lax.axis_index("s") * 2 + lax.axis_index("c")
        for j in range(nch):
            pltpu.sync_copy(idx_hbm.at[wid, j], idx_v)
            pltpu.sync_copy(r_hbm.at[pl.ds(wid * per_w + j * chunk, chunk)],
                            rows_v)
            pltpu.async_copy(rows_v, out_ref.at[idx_v], sem).wait()
    return s


def _p4_body(qs_ref, wq_ref, wk_ref, bq_ref, a_ref):
    qsp = lax.dot_general(qs_ref[0, 0], wq_ref[...], (((1,), (1,)), ((), ())),
                          preferred_element_type=jnp.float32)
    qsp = qsp + bq_ref[...][None, :]
    a_ref[0, 0] = jnp.dot(qsp, wk_ref[...],
                          preferred_element_type=jnp.float32) * SCALE


def _p4(qsel, Wq, Wk, bq):
    return pl.pallas_call(
        _p4_body,
        grid=(B, H),
        in_specs=[
            pl.BlockSpec((1, 1, UP, D), lambda b, h: (b, h, 0, 0)),
            pl.BlockSpec((Dh, D), lambda b, h: (h, 0)),
            pl.BlockSpec((Dh, D), lambda b, h: (h, 0)),
            pl.BlockSpec((Dh,), lambda b, h: (h,)),
        ],
        out_specs=pl.BlockSpec((1, 1, UP, D), lambda b, h: (b, h, 0, 0)),
        out_shape=jax.ShapeDtypeStruct((B, H, UP, D), jnp.float32),
    )(qsel, Wq, Wk, bq)


def _p5_body(a_ref, c_ref, k_ref, v_ref, p_ref, vs_ref,
             acc_ref, m_ref, l_ref, vsum_ref):
    j = pl.program_id(1)

    @pl.when(j == 0)
    def _():
        acc_ref[...] = jnp.zeros_like(acc_ref)
        m_ref[...] = jnp.full_like(m_ref, NEG)
        l_ref[...] = jnp.zeros_like(l_ref)
        vsum_ref[...] = jnp.zeros_like(vsum_ref)

    s = lax.dot_general(a_ref[0], k_ref[0], (((1,), (1,)), ((), ())),
                        preferred_element_type=jnp.float32)
    s = s + c_ref[0]
    mcur = jnp.max(s, axis=1, keepdims=True)
    mnew = jnp.maximum(m_ref[...], mcur)
    alpha = jnp.exp(m_ref[...] - mnew)
    p = jnp.exp(s - mnew)
    l_ref[...] = l_ref[...] * alpha + jnp.sum(p, axis=1, keepdims=True)
    acc_ref[...] = acc_ref[...] * alpha + jnp.dot(
        p, v_ref[0], preferred_element_type=jnp.float32)
    m_ref[...] = mnew
    vsum_ref[...] = vsum_ref[...] + jnp.sum(v_ref[0], axis=0, keepdims=True)

    @pl.when(j == NKB - 1)
    def _():
        p_ref[0] = acc_ref[...] / l_ref[...]
        vs_ref[0] = vsum_ref[...]


def _p5(Acat, c3, k, v):
    return pl.pallas_call(
        _p5_body,
        grid=(B, NKB),
        in_specs=[
            pl.BlockSpec((1, NP, D), lambda b, j: (b, 0, 0)),
            pl.BlockSpec((1, NP, 1), lambda b, j: (b, 0, 0)),
            pl.BlockSpec((1, KBLK, D), lambda b, j: (b, j, 0)),
            pl.BlockSpec((1, KBLK, D), lambda b, j: (b, j, 0)),
        ],
        out_specs=[
            pl.BlockSpec((1, NP, D), lambda b, j: (b, 0, 0)),
            pl.BlockSpec((1, 1, D), lambda b, j: (b, 0, 0)),
        ],
        out_shape=[
            jax.ShapeDtypeStruct((B, NP, D), jnp.float32),
            jax.ShapeDtypeStruct((B, 1, D), jnp.float32),
        ],
        scratch_shapes=[
            pltpu.VMEM((NP, D), jnp.float32),
            pltpu.VMEM((NP, 1), jnp.float32),
            pltpu.VMEM((NP, 1), jnp.float32),
            pltpu.VMEM((1, D), jnp.float32),
        ],
    )(Acat, c3, k, v)


def _p6_body(p_ref, vs_ref, wv_ref, wo_ref, bv_ref, bo_ref, c_ref, base_ref):
    ao = lax.dot_general(p_ref[0, 0], wv_ref[...], (((1,), (1,)), ((), ())),
                         preferred_element_type=jnp.float32)
    vm = lax.dot_general(vs_ref[0] * (1.0 / L), wv_ref[...],
                         (((1,), (1,)), ((), ())),
                         preferred_element_type=jnp.float32)
    delta = ao - vm
    cfull = lax.dot_general(delta, wo_ref[...], (((1,), (1,)), ((), ())),
                            preferred_element_type=jnp.float32)
    zmask = lax.broadcasted_iota(jnp.int32, (UP, 1), 0) < U
    c_ref[0, 0] = jnp.where(zmask, cfull, 0.0)
    bp = lax.dot_general(vm + bv_ref[...][None, :], wo_ref[...],
                         (((1,), (1,)), ((), ())),
                         preferred_element_type=jnp.float32)
    h = pl.program_id(1)

    @pl.when(h == 0)
    def _():
        base_ref[0] = bp + bo_ref[...][None, :]

    @pl.when(h > 0)
    def _():
        base_ref[0] = base_ref[0] + bp


def _p6(P, vs, Wv, Wo, bv, bo):
    return pl.pallas_call(
        _p6_body,
        grid=(B, H),
        in_specs=[
            pl.BlockSpec((1, 1, UP, D), lambda b, h: (b, h, 0, 0)),
            pl.BlockSpec((1, 1, D), lambda b, h: (b, 0, 0)),
            pl.BlockSpec((Dh, D), lambda b, h: (h, 0)),
            pl.BlockSpec((D, Dh), lambda b, h: (0, h)),
            pl.BlockSpec((Dh,), lambda b, h: (h,)),
            pl.BlockSpec((D,), lambda b, h: (0,)),
        ],
        out_specs=[
            pl.BlockSpec((1, 1, UP, D), lambda b, h: (b, h, 0, 0)),
            pl.BlockSpec((1, 1, D), lambda b, h: (b, 0, 0)),
        ],
        out_shape=[
            jax.ShapeDtypeStruct((B, H, UP, D), jnp.float32),
            jax.ShapeDtypeStruct((B, 1, D), jnp.float32),
        ],
    )(P, vs, Wv, Wo, bv, bo)


def _p7_body(ir_ref, ic_ref, c_ref, base_ref, r_ref):
    eq = (ic_ref[0] == ir_ref[0]).astype(jnp.float32)
    r_ref[0] = jnp.dot(eq, c_ref[0],
                       preferred_element_type=jnp.float32) + base_ref[0]


def _p7(gidx_row, gidx_col, Ccat, base):
    return pl.pallas_call(
        _p7_body,
        grid=(B,),
        in_specs=[
            pl.BlockSpec((1, 1, NP), lambda b: (b, 0, 0)),
            pl.BlockSpec((1, NP, 1), lambda b: (b, 0, 0)),
            pl.BlockSpec((1, NP, D), lambda b: (b, 0, 0)),
            pl.BlockSpec((1, 1, D), lambda b: (b, 0, 0)),
        ],
        out_specs=pl.BlockSpec((1, NP, D), lambda b: (b, 0, 0)),
        out_shape=jax.ShapeDtypeStruct((B, NP, D), jnp.float32),
    )(gidx_row, gidx_col, Ccat, base)


def _p8_body(base_ref, out_ref):
    out_ref[...] = jnp.broadcast_to(base_ref[0], out_ref.shape)


def _p8(base):
    return pl.pallas_call(
        _p8_body,
        grid=(B * L // 512,),
        in_specs=[pl.BlockSpec((1, 1, D), lambda j: (j // (L // 512), 0, 0))],
        out_specs=pl.BlockSpec((512, D), lambda j: (j, 0)),
        out_shape=jax.ShapeDtypeStruct((B * L, D), jnp.float32),
    )(base)


def kernel(q, k, v, Wq, bq, Wk, bk, Wv, bv, Wo, bo):
    f32 = jnp.float32
    q, k, v = q.astype(f32), k.astype(f32), v.astype(f32)

    kidx = jax.random.randint(jax.random.key(42), (U,), 0, L)

    kidxp = jnp.concatenate([kidx, jnp.broadcast_to(kidx[:1], (128 - U,))])
    gk = (jnp.arange(B, dtype=jnp.int32)[:, None] * L
          + kidxp[None, :].astype(jnp.int32))
    ksamp_pad = _sc_gather(B * 128, 16)(
        k.reshape(B * L, D), gk.reshape(NW, (B * 128) // NW // 16, 16))
    ksamp = ksamp_pad.reshape(B, 128, D)[:, :U]

    Ks = _p1a(ksamp, Wk, bk)
    M = _p1b(q, Wq.astype(jnp.bfloat16), Ks, bq)
    ti = _p2(M)[:, :, 0, :U]

    ti48 = jnp.concatenate(
        [ti, jnp.broadcast_to(ti[:, :, :1], (B, H, UP - U))], axis=2)
    gidx = (jnp.arange(B, dtype=jnp.int32)[:, None, None] * L
            + ti48)
    gidx_flat = gidx.reshape(NG)

    qsel = _sc_gather(NG, 16)(
        q.reshape(B * L, D),
        gidx_flat.reshape(NW, NG // NW // 16, 16)).reshape(B, H, UP, D)

    A = _p4(qsel, Wq, Wk, bq).reshape(B, NP, D)
    g2 = jnp.einsum('hm,hmd->hd', bk.reshape(H, Dh), Wq.reshape(H, Dh, D))
    c = (jnp.einsum('bhid,hd->bhi', qsel, g2)
         + jnp.einsum('hm,hm->h', bq.reshape(H, Dh),
                      bk.reshape(H, Dh))[None, :, None]) * SCALE
    c3 = c.reshape(B, NP, 1)
    P, vs = _p5(A, c3, k, v)

    C, base = _p6(P.reshape(B, H, UP, D), vs, Wv, Wo, bv, bo)
    gi2 = gidx.reshape(B, NP)
    R = _p7(gi2.reshape(B, 1, NP), gi2.reshape(B, NP, 1),
            C.reshape(B, NP, D), base)

    out0 = _p8(base)
    oref = jax.new_ref(out0)
    _sc_scatter()(R.reshape(NG, D),
                  gidx_flat.reshape(NW, NG // NW // 16, 16), oref)
    return oref[...].reshape(B, L, D)


# speedup vs baseline: 1.9945x; 1.9945x over previous
import functools
import math

import jax
import jax.numpy as jnp
from jax import lax
from jax.experimental import pallas as pl
from jax.experimental.pallas import tpu as pltpu
from jax.experimental.pallas import tpu_sc as plsc

B, L, D, H, FACTOR = 4, 4096, 2048, 16, 5
Dh = D // H
U = max(1, min(FACTOR * int(math.ceil(math.log(L + 1))), L))
SCALE = 1.0 / math.sqrt(Dh)
UP = 48
NP = H * UP
NG = B * NP
NW = 32
LBLK = 512
KBLK = 512
NKB = L // KBLK
NEG = float("-inf")


def _p1a_body(ksamp_ref, wk_ref, bk_ref, ks_ref):
    ks = lax.dot_general(ksamp_ref[0].astype(jnp.bfloat16),
                         wk_ref[...].astype(jnp.bfloat16),
                         (((1,), (1,)), ((), ())),
                         preferred_element_type=jnp.float32)
    ks_ref[0, 0] = (ks + bk_ref[...][None, :]).astype(jnp.bfloat16)


def _p1a(ksamp, Wk, bk):
    return pl.pallas_call(
        _p1a_body,
        grid=(B, H),
        in_specs=[
            pl.BlockSpec((1, U, D), lambda b, h: (b, 0, 0)),
            pl.BlockSpec((Dh, D), lambda b, h: (h, 0)),
            pl.BlockSpec((Dh,), lambda b, h: (h,)),
        ],
        out_specs=pl.BlockSpec((1, 1, U, Dh), lambda b, h: (b, h, 0, 0)),
        out_shape=jax.ShapeDtypeStruct((B, H, U, Dh), jnp.bfloat16),
    )(ksamp, Wk, bk)


def _p1b_body(q_ref, wq_ref, bdt_ref, bq_ref, pm_ref, m_ref):
    qb = lax.dot_general(q_ref[0].astype(jnp.bfloat16), wq_ref[...],
                         (((1,), (1,)), ((), ())),
                         preferred_element_type=jnp.float32)
    qbb = (qb + bq_ref[...][None, :]).astype(jnp.bfloat16)
    st = lax.dot_general(bdt_ref[0], qbb, (((1,), (1,)), ((), ())),
                         preferred_element_type=jnp.float32) * SCALE
    s3 = st.reshape(H, 64, LBLK)
    ssum = jnp.sum(s3, axis=1)
    smax = jnp.max(s3 + pm_ref[...].reshape(H, 64, 1), axis=1)
    m_ref[0] = smax - ssum / jnp.float32(U)


def _p1b(q, Wqb, BDT, bq, pm):
    return pl.pallas_call(
        _p1b_body,
        grid=(B, L // LBLK),
        in_specs=[
            pl.BlockSpec((1, LBLK, D), lambda b, l: (b, l, 0)),
            pl.BlockSpec((D, D), lambda b, l: (0, 0)),
            pl.BlockSpec((1, H * 64, D), lambda b, l: (b, 0, 0)),
            pl.BlockSpec((D,), lambda b, l: (0,)),
            pl.BlockSpec((H * 64, 1), lambda b, l: (0, 0)),
        ],
        out_specs=pl.BlockSpec((1, H, LBLK), lambda b, l: (b, 0, l)),
        out_shape=jax.ShapeDtypeStruct((B, H, L), jnp.float32),
    )(q, Wqb, BDT, bq, pm)


def _p2_body(m_ref, ti_ref):
    x = m_ref[...]
    lin = lax.broadcasted_iota(jnp.int32, (B * H, L), 1)
    lane = lax.broadcasted_iota(jnp.int32, (B * H, 64), 1)
    sel = jnp.zeros((B * H, 64), jnp.int32)
    for i in range(U):
        mx = jnp.max(x, axis=1, keepdims=True)
        idx = jnp.min(jnp.where(x == mx, lin, L), axis=1, keepdims=True)
        sel = jnp.where(lane == i, idx, sel)
        x = jnp.where(lin == idx, NEG, x)
    ti_ref[...] = sel


def _p2(M2):
    return pl.pallas_call(
        _p2_body,
        in_specs=[pl.BlockSpec((B * H, L), lambda: (0, 0))],
        out_specs=pl.BlockSpec((B * H, 64), lambda: (0, 0)),
        out_shape=jax.ShapeDtypeStruct((B * H, 64), jnp.int32),
    )(M2)


def _sc_gather(nrows, chunk):
    per_w = nrows // NW
    nch = per_w // chunk
    mesh = plsc.VectorSubcoreMesh(core_axis_name="c", subcore_axis_name="s")

    @functools.partial(
        pl.kernel, mesh=mesh,
        out_type=jax.ShapeDtypeStruct((nrows, D), jnp.float32),
        scratch_types=[
            pltpu.VMEM((chunk,), jnp.int32),
            pltpu.VMEM((chunk, D), jnp.float32),
            pltpu.SemaphoreType.DMA,
        ],
    )
    def g(table_hbm, idx_hbm, out_hbm, idx_v, rows_v, sem):
        wid = lax.axis_index("s") * 2 + lax.axis_index("c")
        for j in range(nch):
            pltpu.sync_copy(idx_hbm.at[wid, j], idx_v)
            pltpu.async_copy(table_hbm.at[idx_v], rows_v, sem).wait()
            pltpu.sync_copy(rows_v, out_hbm.at[pl.ds(wid * per_w + j * chunk,
                                                     chunk)])
    return g


def _sc_scatter():
    per_w = NG // NW
    chunk = 16
    nch = per_w // chunk
    mesh = plsc.VectorSubcoreMesh(core_axis_name="c", subcore_axis_name="s")

    @functools.partial(
        pl.kernel, mesh=mesh,
        out_type=(),
        scratch_types=[
            pltpu.VMEM((chunk,), jnp.int32),
            pltpu.VMEM((chunk, D), jnp.float32),
            pltpu.SemaphoreType.DMA,
        ],
    )
    def s(r_hbm, idx_hbm, out_ref, idx_v, rows_v, sem):
        wid = lax.axis_index("s") * 2 + lax.axis_index("c")
        for j in range(nch):
            pltpu.sync_copy(idx_hbm.at[wid, j], idx_v)
            pltpu.sync_copy(r_hbm.at[pl.ds(wid * per_w + j * chunk, chunk)],
                            rows_v)
            pltpu.async_copy(rows_v, out_ref.at[idx_v], sem).wait()
    return s


def _p4_body(qs_ref, wq_ref, wk_ref, bq_ref, a_ref):
    qsp = lax.dot_general(qs_ref[0, 0], wq_ref[...], (((1,), (1,)), ((), ())),
                          preferred_element_type=jnp.float32)
    qsp = qsp + bq_ref[...][None, :]
    a_ref[0, 0] = (jnp.dot(qsp, wk_ref[...],
                           preferred_element_type=jnp.float32)
                   * SCALE).astype(jnp.bfloat16)


def _p4(qsel, Wq, Wk, bq):
    return pl.pallas_call(
        _p4_body,
        grid=(B, H),
        in_specs=[
            pl.BlockSpec((1, 1, UP, D), lambda b, h: (b, h, 0, 0)),
            pl.BlockSpec((Dh, D), lambda b, h: (h, 0)),
            pl.BlockSpec((Dh, D), lambda b, h: (h, 0)),
            pl.BlockSpec((Dh,), lambda b, h: (h,)),
        ],
        out_specs=pl.BlockSpec((1, 1, UP, D), lambda b, h: (b, h, 0, 0)),
        out_shape=jax.ShapeDtypeStruct((B, H, UP, D), jnp.bfloat16),
    )(qsel, Wq, Wk, bq)


def _p5_body(a_ref, c_ref, k_ref, v_ref, p_ref, vs_ref,
             acc_ref, m_ref, l_ref, vsum_ref):
    j = pl.program_id(1)

    @pl.when(j == 0)
    def _():
        acc_ref[...] = jnp.zeros_like(acc_ref)
        m_ref[...] = jnp.full_like(m_ref, NEG)
        l_ref[...] = jnp.zeros_like(l_ref)
        vsum_ref[...] = jnp.zeros_like(vsum_ref)

    s = lax.dot_general(a_ref[0], k_ref[0].astype(jnp.bfloat16),
                        (((1,), (1,)), ((), ())),
                        preferred_element_type=jnp.float32)
    s = s + c_ref[0]
    mcur = jnp.max(s, axis=1, keepdims=True)
    mnew = jnp.maximum(m_ref[...], mcur)
    alpha = jnp.exp(m_ref[...] - mnew)
    p = jnp.exp(s - mnew)
    l_ref[...] = l_ref[...] * alpha + jnp.sum(p, axis=1, keepdims=True)
    acc_ref[...] = acc_ref[...] * alpha + jnp.dot(
        p.astype(jnp.bfloat16), v_ref[0].astype(jnp.bfloat16),
        preferred_element_type=jnp.float32)
    m_ref[...] = mnew
    vsum_ref[...] = vsum_ref[...] + jnp.sum(v_ref[0], axis=0, keepdims=True)

    @pl.when(j == NKB - 1)
    def _():
        p_ref[0] = acc_ref[...] / l_ref[...]
        vs_ref[0] = vsum_ref[...]


def _p5(Acat, c3, k, v):
    return pl.pallas_call(
        _p5_body,
        grid=(B, NKB),
        in_specs=[
            pl.BlockSpec((1, NP, D), lambda b, j: (b, 0, 0)),
            pl.BlockSpec((1, NP, 1), lambda b, j: (b, 0, 0)),
            pl.BlockSpec((1, KBLK, D), lambda b, j: (b, j, 0)),
            pl.BlockSpec((1, KBLK, D), lambda b, j: (b, j, 0)),
        ],
        out_specs=[
            pl.BlockSpec((1, NP, D), lambda b, j: (b, 0, 0)),
            pl.BlockSpec((1, 1, D), lambda b, j: (b, 0, 0)),
        ],
        out_shape=[
            jax.ShapeDtypeStruct((B, NP, D), jnp.float32),
            jax.ShapeDtypeStruct((B, 1, D), jnp.float32),
        ],
        scratch_shapes=[
            pltpu.VMEM((NP, D), jnp.float32),
            pltpu.VMEM((NP, 1), jnp.float32),
            pltpu.VMEM((NP, 1), jnp.float32),
            pltpu.VMEM((1, D), jnp.float32),
        ],
    )(Acat, c3, k, v)


def _p6_body(p_ref, vs_ref, wv_ref, wo_ref, bv_ref, bo_ref, c_ref, base_ref):
    ao = lax.dot_general(p_ref[0, 0], wv_ref[...], (((1,), (1,)), ((), ())),
                         preferred_element_type=jnp.float32)
    vm = lax.dot_general(vs_ref[0] * (1.0 / L), wv_ref[...],
                         (((1,), (1,)), ((), ())),
                         preferred_element_type=jnp.float32)
    delta = ao - vm
    cfull = lax.dot_general(delta, wo_ref[...], (((1,), (1,)), ((), ())),
                            preferred_element_type=jnp.float32)
    zmask = lax.broadcasted_iota(jnp.int32, (UP, 1), 0) < U
    c_ref[0, 0] = jnp.where(zmask, cfull, 0.0)
    bp = lax.dot_general(vm + bv_ref[...][None, :], wo_ref[...],
                         (((1,), (1,)), ((), ())),
                         preferred_element_type=jnp.float32)
    h = pl.program_id(1)

    @pl.when(h == 0)
    def _():
        base_ref[0] = bp + bo_ref[...][None, :]

    @pl.when(h > 0)
    def _():
        base_ref[0] = base_ref[0] + bp


def _p6(P, vs, Wv, Wo, bv, bo):
    return pl.pallas_call(
        _p6_body,
        grid=(B, H),
        in_specs=[
            pl.BlockSpec((1, 1, UP, D), lambda b, h: (b, h, 0, 0)),
            pl.BlockSpec((1, 1, D), lambda b, h: (b, 0, 0)),
            pl.BlockSpec((Dh, D), lambda b, h: (h, 0)),
            pl.BlockSpec((D, Dh), lambda b, h: (0, h)),
            pl.BlockSpec((Dh,), lambda b, h: (h,)),
            pl.BlockSpec((D,), lambda b, h: (0,)),
        ],
        out_specs=[
            pl.BlockSpec((1, 1, UP, D), lambda b, h: (b, h, 0, 0)),
            pl.BlockSpec((1, 1, D), lambda b, h: (b, 0, 0)),
        ],
        out_shape=[
            jax.ShapeDtypeStruct((B, H, UP, D), jnp.float32),
            jax.ShapeDtypeStruct((B, 1, D), jnp.float32),
        ],
    )(P, vs, Wv, Wo, bv, bo)


def _p7_body(ir_ref, ic_ref, c_ref, base_ref, r_ref):
    eq = (ic_ref[0] == ir_ref[0]).astype(jnp.float32)
    r_ref[0] = jnp.dot(eq, c_ref[0],
                       preferred_element_type=jnp.float32) + base_ref[0]


def _p7(gidx_row, gidx_col, Ccat, base):
    return pl.pallas_call(
        _p7_body,
        grid=(B,),
        in_specs=[
            pl.BlockSpec((1, 1, NP), lambda b: (b, 0, 0)),
            pl.BlockSpec((1, NP, 1), lambda b: (b, 0, 0)),
            pl.BlockSpec((1, NP, D), lambda b: (b, 0, 0)),
            pl.BlockSpec((1, 1, D), lambda b: (b, 0, 0)),
        ],
        out_specs=pl.BlockSpec((1, NP, D), lambda b: (b, 0, 0)),
        out_shape=jax.ShapeDtypeStruct((B, NP, D), jnp.float32),
    )(gidx_row, gidx_col, Ccat, base)


def _p8_body(base_ref, out_ref):
    out_ref[...] = jnp.broadcast_to(base_ref[0], out_ref.shape)


def _p8(base):
    return pl.pallas_call(
        _p8_body,
        grid=(B * L // 512,),
        in_specs=[pl.BlockSpec((1, 1, D), lambda j: (j // (L // 512), 0, 0))],
        out_specs=pl.BlockSpec((512, D), lambda j: (j, 0)),
        out_shape=jax.ShapeDtypeStruct((B * L, D), jnp.float32),
    )(base)


def kernel(q, k, v, Wq, bq, Wk, bk, Wv, bv, Wo, bo):
    f32 = jnp.float32
    q, k, v = q.astype(f32), k.astype(f32), v.astype(f32)

    kidx = jax.random.randint(jax.random.key(42), (U,), 0, L)

    kidxp = jnp.concatenate([kidx, jnp.broadcast_to(kidx[:1], (128 - U,))])
    gk = (jnp.arange(B, dtype=jnp.int32)[:, None] * L
          + kidxp[None, :].astype(jnp.int32))
    ksamp_pad = _sc_gather(B * 128, 16)(
        k.reshape(B * L, D), gk.reshape(NW, (B * 128) // NW // 16, 16))
    ksamp = ksamp_pad.reshape(B, 128, D)[:, :U]

    Ks = _p1a(ksamp, Wk, bk)
    Ksp = jnp.concatenate(
        [Ks, jnp.zeros((B, H, 64 - U, Dh), jnp.bfloat16)], axis=2)
    BDT = jnp.einsum('bhum,gh->bguhm', Ksp,
                     jnp.eye(H, dtype=jnp.bfloat16)).reshape(B, H * 64, D)
    pm = jnp.where(jnp.arange(64)[None, :] < U, 0.0, -jnp.inf)
    pm = jnp.broadcast_to(pm, (H, 64)).reshape(H * 64, 1).astype(jnp.float32)
    M = _p1b(q, Wq.astype(jnp.bfloat16), BDT, bq, pm)
    ti = _p2(M.reshape(B * H, L)).reshape(B, H, 64)[:, :, :U]

    ti48 = jnp.concatenate(
        [ti, jnp.broadcast_to(ti[:, :, :1], (B, H, UP - U))], axis=2)
    gidx = (jnp.arange(B, dtype=jnp.int32)[:, None, None] * L
            + ti48)
    gidx_flat = gidx.reshape(NG)

    qsel = _sc_gather(NG, 16)(
        q.reshape(B * L, D),
        gidx_flat.reshape(NW, NG // NW // 16, 16)).reshape(B, H, UP, D)

    A = _p4(qsel, Wq, Wk, bq).reshape(B, NP, D)
    g2 = jnp.einsum('hm,hmd->hd', bk.reshape(H, Dh), Wq.reshape(H, Dh, D))
    c = (jnp.einsum('bhid,hd->bhi', qsel, g2)
         + jnp.einsum('hm,hm->h', bq.reshape(H, Dh),
                      bk.reshape(H, Dh))[None, :, None]) * SCALE
    c3 = c.reshape(B, NP, 1)
    P, vs = _p5(A, c3, k, v)

    C, base = _p6(P.reshape(B, H, UP, D), vs, Wv, Wo, bv, bo)
    gi2 = gidx.reshape(B, NP)
    R = _p7(gi2.reshape(B, 1, NP), gi2.reshape(B, NP, 1),
            C.reshape(B, NP, D), base)

    out0 = _p8(base)
    oref = jax.new_ref(out0)
    _sc_scatter()(R.reshape(NG, D),
                  gidx_flat.reshape(NW, NG // NW // 16, 16), oref)
    return oref[...].reshape(B, L, D)


# speedup vs baseline: 2.0044x; 1.0050x over previous
import functools
import math

import jax
import jax.numpy as jnp
from jax import lax
from jax.experimental import pallas as pl
from jax.experimental.pallas import tpu as pltpu
from jax.experimental.pallas import tpu_sc as plsc

B, L, D, H, FACTOR = 4, 4096, 2048, 16, 5
Dh = D // H
U = max(1, min(FACTOR * int(math.ceil(math.log(L + 1))), L))
SCALE = 1.0 / math.sqrt(Dh)
UP = 48
NP = H * UP
NG = B * NP
NW = 32
LBLK = 1024
KBLK = 512
NKB = L // KBLK
NEG = float("-inf")


def _p1a_body(ksamp_ref, wk_ref, bk_ref, ks_ref):
    ks = lax.dot_general(ksamp_ref[0].astype(jnp.bfloat16),
                         wk_ref[...].astype(jnp.bfloat16),
                         (((1,), (1,)), ((), ())),
                         preferred_element_type=jnp.float32)
    ks_ref[0, 0] = (ks + bk_ref[...][None, :]).astype(jnp.bfloat16)


def _p1a(ksamp, Wk, bk):
    return pl.pallas_call(
        _p1a_body,
        grid=(B, H),
        in_specs=[
            pl.BlockSpec((1, U, D), lambda b, h: (b, 0, 0)),
            pl.BlockSpec((Dh, D), lambda b, h: (h, 0)),
            pl.BlockSpec((Dh,), lambda b, h: (h,)),
        ],
        out_specs=pl.BlockSpec((1, 1, U, Dh), lambda b, h: (b, h, 0, 0)),
        out_shape=jax.ShapeDtypeStruct((B, H, U, Dh), jnp.bfloat16),
    )(ksamp, Wk, bk)


def _p1b_body(q_ref, wq_ref, bdt_ref, bq_ref, pm_ref, m_ref):
    qb = lax.dot_general(q_ref[0].astype(jnp.bfloat16), wq_ref[...],
                         (((1,), (1,)), ((), ())),
                         preferred_element_type=jnp.float32)
    qbb = (qb + bq_ref[...][None, :]).astype(jnp.bfloat16)
    st = lax.dot_general(bdt_ref[0], qbb, (((1,), (1,)), ((), ())),
                         preferred_element_type=jnp.float32) * SCALE
    s3 = st.reshape(H, 64, LBLK)
    ssum = jnp.sum(s3, axis=1)
    smax = jnp.max(s3 + pm_ref[...].reshape(H, 64, 1), axis=1)
    m_ref[0] = smax - ssum / jnp.float32(U)


def _p1b(q, Wqb, BDT, bq, pm):
    return pl.pallas_call(
        _p1b_body,
        grid=(B, L // LBLK),
        in_specs=[
            pl.BlockSpec((1, LBLK, D), lambda b, l: (b, l, 0)),
            pl.BlockSpec((D, D), lambda b, l: (0, 0)),
            pl.BlockSpec((1, H * 64, D), lambda b, l: (b, 0, 0)),
            pl.BlockSpec((D,), lambda b, l: (0,)),
            pl.BlockSpec((H * 64, 1), lambda b, l: (0, 0)),
        ],
        out_specs=pl.BlockSpec((1, H, LBLK), lambda b, l: (b, 0, l)),
        out_shape=jax.ShapeDtypeStruct((B, H, L), jnp.float32),
    )(q, Wqb, BDT, bq, pm)


def _p2_body(m_ref, ti_ref):
    x = m_ref[...]
    lin = lax.broadcasted_iota(jnp.int32, (B * H, L), 1)
    lane = lax.broadcasted_iota(jnp.int32, (B * H, 64), 1)
    sel = jnp.zeros((B * H, 64), jnp.int32)
    for i in range(U):
        mx = jnp.max(x, axis=1, keepdims=True)
        idx = jnp.min(jnp.where(x == mx, lin, L), axis=1, keepdims=True)
        sel = jnp.where(lane == i, idx, sel)
        x = jnp.where(lin == idx, NEG, x)
    ti_ref[...] = sel


def _p2(M2):
    return pl.pallas_call(
        _p2_body,
        in_specs=[pl.BlockSpec((B * H, L), lambda: (0, 0))],
        out_specs=pl.BlockSpec((B * H, 64), lambda: (0, 0)),
        out_shape=jax.ShapeDtypeStruct((B * H, 64), jnp.int32),
    )(M2)


def _sc_gather(nrows, chunk):
    per_w = nrows // NW
    nch = per_w // chunk
    mesh = plsc.VectorSubcoreMesh(core_axis_name="c", subcore_axis_name="s")

    @functools.partial(
        pl.kernel, mesh=mesh,
        out_type=jax.ShapeDtypeStruct((nrows, D), jnp.float32),
        scratch_types=[
            pltpu.VMEM((chunk,), jnp.int32),
            pltpu.VMEM((chunk, D), jnp.float32),
            pltpu.SemaphoreType.DMA,
        ],
    )
    def g(table_hbm, idx_hbm, out_hbm, idx_v, rows_v, sem):
        wid = lax.axis_index("s") * 2 + lax.axis_index("c")
        for j in range(nch):
            pltpu.sync_copy(idx_hbm.at[wid, j], idx_v)
            pltpu.async_copy(table_hbm.at[idx_v], rows_v, sem).wait()
            pltpu.sync_copy(rows_v, out_hbm.at[pl.ds(wid * per_w + j * chunk,
                                                     chunk)])
    return g


def _sc_scatter():
    per_w = NG // NW
    chunk = 16
    nch = per_w // chunk
    mesh = plsc.VectorSubcoreMesh(core_axis_name="c", subcore_axis_name="s")

    @functools.partial(
        pl.kernel, mesh=mesh,
        out_type=(),
        scratch_types=[
            pltpu.VMEM((chunk,), jnp.int32),
            pltpu.VMEM((chunk, D), jnp.float32),
            pltpu.SemaphoreType.DMA,
        ],
    )
    def s(r_hbm, idx_hbm, out_ref, idx_v, rows_v, sem):
        wid = lax.axis_index("s") * 2 + lax.axis_index("c")
        for j in range(nch):
            pltpu.sync_copy(idx_hbm.at[wid, j], idx_v)
            pltpu.sync_copy(r_hbm.at[pl.ds(wid * per_w + j * chunk, chunk)],
                            rows_v)
            pltpu.async_copy(rows_v, out_ref.at[idx_v], sem).wait()
    return s


def _p4_body(qs_ref, wq_ref, wk_ref, bq_ref, a_ref):
    qsp = lax.dot_general(qs_ref[0, 0], wq_ref[...], (((1,), (1,)), ((), ())),
                          preferred_element_type=jnp.float32)
    qsp = qsp + bq_ref[...][None, :]
    a_ref[0, 0] = (jnp.dot(qsp, wk_ref[...],
                           preferred_element_type=jnp.float32)
                   * SCALE).astype(jnp.bfloat16)


def _p4(qsel, Wq, Wk, bq):
    return pl.pallas_call(
        _p4_body,
        grid=(B, H),
        in_specs=[
            pl.BlockSpec((1, 1, UP, D), lambda b, h: (b, h, 0, 0)),
            pl.BlockSpec((Dh, D), lambda b, h: (h, 0)),
            pl.BlockSpec((Dh, D), lambda b, h: (h, 0)),
            pl.BlockSpec((Dh,), lambda b, h: (h,)),
        ],
        out_specs=pl.BlockSpec((1, 1, UP, D), lambda b, h: (b, h, 0, 0)),
        out_shape=jax.ShapeDtypeStruct((B, H, UP, D), jnp.bfloat16),
    )(qsel, Wq, Wk, bq)


def _p5_body(a_ref, c_ref, k_ref, v_ref, p_ref, vs_ref,
             acc_ref, m_ref, l_ref, vsum_ref):
    j = pl.program_id(1)

    @pl.when(j == 0)
    def _():
        acc_ref[...] = jnp.zeros_like(acc_ref)
        m_ref[...] = jnp.full_like(m_ref, NEG)
        l_ref[...] = jnp.zeros_like(l_ref)
        vsum_ref[...] = jnp.zeros_like(vsum_ref)

    s = lax.dot_general(a_ref[0], k_ref[0].astype(jnp.bfloat16),
                        (((1,), (1,)), ((), ())),
                        preferred_element_type=jnp.float32)
    s = s + c_ref[0]
    mcur = jnp.max(s, axis=1, keepdims=True)
    mnew = jnp.maximum(m_ref[...], mcur)
    alpha = jnp.exp(m_ref[...] - mnew)
    p = jnp.exp(s - mnew)
    l_ref[...] = l_ref[...] * alpha + jnp.sum(p, axis=1, keepdims=True)
    acc_ref[...] = acc_ref[...] * alpha + jnp.dot(
        p.astype(jnp.bfloat16), v_ref[0].astype(jnp.bfloat16),
        preferred_element_type=jnp.float32)
    m_ref[...] = mnew
    vsum_ref[...] = vsum_ref[...] + jnp.sum(v_ref[0], axis=0, keepdims=True)

    @pl.when(j == NKB - 1)
    def _():
        p_ref[0] = acc_ref[...] / l_ref[...]
        vs_ref[0] = vsum_ref[...]


def _p5(Acat, c3, k, v):
    return pl.pallas_call(
        _p5_body,
        grid=(B, NKB),
        in_specs=[
            pl.BlockSpec((1, NP, D), lambda b, j: (b, 0, 0)),
            pl.BlockSpec((1, NP, 1), lambda b, j: (b, 0, 0)),
            pl.BlockSpec((1, KBLK, D), lambda b, j: (b, j, 0)),
            pl.BlockSpec((1, KBLK, D), lambda b, j: (b, j, 0)),
        ],
        out_specs=[
            pl.BlockSpec((1, NP, D), lambda b, j: (b, 0, 0)),
            pl.BlockSpec((1, 1, D), lambda b, j: (b, 0, 0)),
        ],
        out_shape=[
            jax.ShapeDtypeStruct((B, NP, D), jnp.float32),
            jax.ShapeDtypeStruct((B, 1, D), jnp.float32),
        ],
        scratch_shapes=[
            pltpu.VMEM((NP, D), jnp.float32),
            pltpu.VMEM((NP, 1), jnp.float32),
            pltpu.VMEM((NP, 1), jnp.float32),
            pltpu.VMEM((1, D), jnp.float32),
        ],
    )(Acat, c3, k, v)


def _p6_body(p_ref, vs_ref, wv_ref, wo_ref, bv_ref, bo_ref, c_ref, base_ref):
    ao = lax.dot_general(p_ref[0, 0], wv_ref[...], (((1,), (1,)), ((), ())),
                         preferred_element_type=jnp.float32)
    vm = lax.dot_general(vs_ref[0] * (1.0 / L), wv_ref[...],
                         (((1,), (1,)), ((), ())),
                         preferred_element_type=jnp.float32)
    delta = ao - vm
    cfull = lax.dot_general(delta, wo_ref[...], (((1,), (1,)), ((), ())),
                            preferred_element_type=jnp.float32)
    zmask = lax.broadcasted_iota(jnp.int32, (UP, 1), 0) < U
    c_ref[0, 0] = jnp.where(zmask, cfull, 0.0)
    bp = lax.dot_general(vm + bv_ref[...][None, :], wo_ref[...],
                         (((1,), (1,)), ((), ())),
                         preferred_element_type=jnp.float32)
    h = pl.program_id(1)

    @pl.when(h == 0)
    def _():
        base_ref[0] = bp + bo_ref[...][None, :]

    @pl.when(h > 0)
    def _():
        base_ref[0] = base_ref[0] + bp


def _p6(P, vs, Wv, Wo, bv, bo):
    return pl.pallas_call(
        _p6_body,
        grid=(B, H),
        in_specs=[
            pl.BlockSpec((1, 1, UP, D), lambda b, h: (b, h, 0, 0)),
            pl.BlockSpec((1, 1, D), lambda b, h: (b, 0, 0)),
            pl.BlockSpec((Dh, D), lambda b, h: (h, 0)),
            pl.BlockSpec((D, Dh), lambda b, h: (0, h)),
            pl.BlockSpec((Dh,), lambda b, h: (h,)),
            pl.BlockSpec((D,), lambda b, h: (0,)),
        ],
        out_specs=[
            pl.BlockSpec((1, 1, UP, D), lambda b, h: (b, h, 0, 0)),
            pl.BlockSpec((1, 1, D), lambda b, h: (b, 0, 0)),
        ],
        out_shape=[
            jax.ShapeDtypeStruct((B, H, UP, D), jnp.float32),
            jax.ShapeDtypeStruct((B, 1, D), jnp.float32),
        ],
    )(P, vs, Wv, Wo, bv, bo)


def _p7_body(ir_ref, ic_ref, c_ref, base_ref, r_ref):
    eq = (ic_ref[0] == ir_ref[0]).astype(jnp.float32)
    r_ref[0] = jnp.dot(eq, c_ref[0],
                       preferred_element_type=jnp.float32) + base_ref[0]


def _p7(gidx_row, gidx_col, Ccat, base):
    return pl.pallas_call(
        _p7_body,
        grid=(B,),
        in_specs=[
            pl.BlockSpec((1, 1, NP), lambda b: (b, 0, 0)),
            pl.BlockSpec((1, NP, 1), lambda b: (b, 0, 0)),
            pl.BlockSpec((1, NP, D), lambda b: (b, 0, 0)),
            pl.BlockSpec((1, 1, D), lambda b: (b, 0, 0)),
        ],
        out_specs=pl.BlockSpec((1, NP, D), lambda b: (b, 0, 0)),
        out_shape=jax.ShapeDtypeStruct((B, NP, D), jnp.float32),
    )(gidx_row, gidx_col, Ccat, base)


def _p8_body(base_ref, out_ref):
    out_ref[...] = jnp.broadcast_to(base_ref[0], out_ref.shape)


def _p8(base):
    return pl.pallas_call(
        _p8_body,
        grid=(B * L // 512,),
        in_specs=[pl.BlockSpec((1, 1, D), lambda j: (j // (L // 512), 0, 0))],
        out_specs=pl.BlockSpec((512, D), lambda j: (j, 0)),
        out_shape=jax.ShapeDtypeStruct((B * L, D), jnp.float32),
    )(base)


def kernel(q, k, v, Wq, bq, Wk, bk, Wv, bv, Wo, bo):
    f32 = jnp.float32
    q, k, v = q.astype(f32), k.astype(f32), v.astype(f32)

    kidx = jax.random.randint(jax.random.key(42), (U,), 0, L)

    kidxp = jnp.concatenate([kidx, jnp.broadcast_to(kidx[:1], (128 - U,))])
    gk = (jnp.arange(B, dtype=jnp.int32)[:, None] * L
          + kidxp[None, :].astype(jnp.int32))
    ksamp_pad = _sc_gather(B * 128, 16)(
        k.reshape(B * L, D), gk.reshape(NW, (B * 128) // NW // 16, 16))
    ksamp = ksamp_pad.reshape(B, 128, D)[:, :U]

    Ks = _p1a(ksamp, Wk, bk)
    Ksp = jnp.concatenate(
        [Ks, jnp.zeros((B, H, 64 - U, Dh), jnp.bfloat16)], axis=2)
    BDT = jnp.einsum('bhum,gh->bguhm', Ksp,
                     jnp.eye(H, dtype=jnp.bfloat16)).reshape(B, H * 64, D)
    pm = jnp.where(jnp.arange(64)[None, :] < U, 0.0, -jnp.inf)
    pm = jnp.broadcast_to(pm, (H, 64)).reshape(H * 64, 1).astype(jnp.float32)
    M = _p1b(q, Wq.astype(jnp.bfloat16), BDT, bq, pm)
    ti = _p2(M.reshape(B * H, L)).reshape(B, H, 64)[:, :, :U]

    ti48 = jnp.concatenate(
        [ti, jnp.broadcast_to(ti[:, :, :1], (B, H, UP - U))], axis=2)
    gidx = (jnp.arange(B, dtype=jnp.int32)[:, None, None] * L
            + ti48)
    gidx_flat = gidx.reshape(NG)

    qsel = _sc_gather(NG, 16)(
        q.reshape(B * L, D),
        gidx_flat.reshape(NW, NG // NW // 16, 16)).reshape(B, H, UP, D)

    A = _p4(qsel, Wq, Wk, bq).reshape(B, NP, D)
    g2 = jnp.einsum('hm,hmd->hd', bk.reshape(H, Dh), Wq.reshape(H, Dh, D))
    c = (jnp.einsum('bhid,hd->bhi', qsel, g2)
         + jnp.einsum('hm,hm->h', bq.reshape(H, Dh),
                      bk.reshape(H, Dh))[None, :, None]) * SCALE
    c3 = c.reshape(B, NP, 1)
    P, vs = _p5(A, c3, k, v)

    C, base = _p6(P.reshape(B, H, UP, D), vs, Wv, Wo, bv, bo)
    gi2 = gidx.reshape(B, NP)
    R = _p7(gi2.reshape(B, 1, NP), gi2.reshape(B, NP, 1),
            C.reshape(B, NP, D), base)

    out0 = _p8(base)
    oref = jax.new_ref(out0)
    _sc_scatter()(R.reshape(NG, D),
                  gidx_flat.reshape(NW, NG // NW // 16, 16), oref)
    return oref[...].reshape(B, L, D)


# speedup vs baseline: 2.1717x; 1.0834x over previous
import functools
import math

import jax
import jax.numpy as jnp
from jax import lax
from jax.experimental import pallas as pl
from jax.experimental.pallas import tpu as pltpu
from jax.experimental.pallas import tpu_sc as plsc

B, L, D, H, FACTOR = 4, 4096, 2048, 16, 5
Dh = D // H
U = max(1, min(FACTOR * int(math.ceil(math.log(L + 1))), L))
SCALE = 1.0 / math.sqrt(Dh)
UP = 48
NP = H * UP
NG = B * NP
NW = 32
LBLK = 1024
KBLK = 512
NKB = L // KBLK
NEG = float("-inf")


def _p1a_body(ksamp_ref, wk_ref, bk_ref, bdt_ref):
    ksb = ksamp_ref[0].astype(jnp.bfloat16)
    wkb = wk_ref[...].astype(jnp.bfloat16)
    bdt_ref[0] = jnp.zeros((H * 64, D), jnp.bfloat16)
    zpad = jnp.zeros((64 - U, Dh), jnp.bfloat16)
    for h in range(H):
        ks = lax.dot_general(ksb, wkb[h * Dh:(h + 1) * Dh, :],
                             (((1,), (1,)), ((), ())),
                             preferred_element_type=jnp.float32)
        ksx = (ks + bk_ref[h * Dh:(h + 1) * Dh][None, :]).astype(jnp.bfloat16)
        bdt_ref[0, h * 64:h * 64 + 64, h * Dh:(h + 1) * Dh] = (
            jnp.concatenate([ksx, zpad], axis=0))


def _p1a(ksamp, Wk, bk):
    return pl.pallas_call(
        _p1a_body,
        grid=(B,),
        in_specs=[
            pl.BlockSpec((1, U, D), lambda b: (b, 0, 0)),
            pl.BlockSpec((D, D), lambda b: (0, 0)),
            pl.BlockSpec((D,), lambda b: (0,)),
        ],
        out_specs=pl.BlockSpec((1, H * 64, D), lambda b: (b, 0, 0)),
        out_shape=jax.ShapeDtypeStruct((B, H * 64, D), jnp.bfloat16),
    )(ksamp, Wk, bk)


def _p1b_body(q_ref, wq_ref, bdt_ref, bq_ref, pm_ref, m_ref):
    qb = lax.dot_general(q_ref[0].astype(jnp.bfloat16), wq_ref[...],
                         (((1,), (1,)), ((), ())),
                         preferred_element_type=jnp.float32)
    qbb = (qb + bq_ref[...][None, :]).astype(jnp.bfloat16)
    st = lax.dot_general(bdt_ref[0], qbb, (((1,), (1,)), ((), ())),
                         preferred_element_type=jnp.float32) * SCALE
    s3 = st.reshape(H, 64, LBLK)
    ssum = jnp.sum(s3, axis=1)
    smax = jnp.max(s3 + pm_ref[...].reshape(H, 64, 1), axis=1)
    m_ref[0] = smax - ssum / jnp.float32(U)


def _p1b(q, Wqb, BDT, bq, pm):
    return pl.pallas_call(
        _p1b_body,
        grid=(B, L // LBLK),
        in_specs=[
            pl.BlockSpec((1, LBLK, D), lambda b, l: (b, l, 0)),
            pl.BlockSpec((D, D), lambda b, l: (0, 0)),
            pl.BlockSpec((1, H * 64, D), lambda b, l: (b, 0, 0)),
            pl.BlockSpec((D,), lambda b, l: (0,)),
            pl.BlockSpec((H * 64, 1), lambda b, l: (0, 0)),
        ],
        out_specs=pl.BlockSpec((1, H, LBLK), lambda b, l: (b, 0, l)),
        out_shape=jax.ShapeDtypeStruct((B, H, L), jnp.float32),
    )(q, Wqb, BDT, bq, pm)


def _p2_body(m_ref, ti_ref):
    x = m_ref[...]
    lin = lax.broadcasted_iota(jnp.int32, (B * H, L), 1)
    lane = lax.broadcasted_iota(jnp.int32, (B * H, 64), 1)
    sel = jnp.zeros((B * H, 64), jnp.int32)
    for i in range(U):
        mx = jnp.max(x, axis=1, keepdims=True)
        idx = jnp.min(jnp.where(x == mx, lin, L), axis=1, keepdims=True)
        sel = jnp.where(lane == i, idx, sel)
        x = jnp.where(lin == idx, NEG, x)
    ti_ref[...] = sel


def _p2(M2):
    return pl.pallas_call(
        _p2_body,
        in_specs=[pl.BlockSpec((B * H, L), lambda: (0, 0))],
        out_specs=pl.BlockSpec((B * H, 64), lambda: (0, 0)),
        out_shape=jax.ShapeDtypeStruct((B * H, 64), jnp.int32),
    )(M2)


def _sc_gather(nrows, chunk):
    per_w = nrows // NW
    nch = per_w // chunk
    mesh = plsc.VectorSubcoreMesh(core_axis_name="c", subcore_axis_name="s")

    @functools.partial(
        pl.kernel, mesh=mesh,
        out_type=jax.ShapeDtypeStruct((nrows, D), jnp.float32),
        scratch_types=[
            pltpu.VMEM((chunk,), jnp.int32),
            pltpu.VMEM((chunk, D), jnp.float32),
            pltpu.SemaphoreType.DMA,
        ],
    )
    def g(table_hbm, idx_hbm, out_hbm, idx_v, rows_v, sem):
        wid = lax.axis_index("s") * 2 + lax.axis_index("c")
        for j in range(nch):
            pltpu.sync_copy(idx_hbm.at[wid, j], idx_v)
            pltpu.async_copy(table_hbm.at[idx_v], rows_v, sem).wait()
            pltpu.sync_copy(rows_v, out_hbm.at[pl.ds(wid * per_w + j * chunk,
                                                     chunk)])
    return g


def _sc_scatter():
    per_w = NG // NW
    chunk = 16
    nch = per_w // chunk
    mesh = plsc.VectorSubcoreMesh(core_axis_name="c", subcore_axis_name="s")

    @functools.partial(
        pl.kernel, mesh=mesh,
        out_type=(),
        scratch_types=[
            pltpu.VMEM((chunk,), jnp.int32),
            pltpu.VMEM((chunk, D), jnp.float32),
            pltpu.SemaphoreType.DMA,
        ],
    )
    def s(r_hbm, idx_hbm, out_ref, idx_v, rows_v, sem):
        wid = lax.axis_index("s") * 2 + lax.axis_index("c")
        for j in range(nch):
            pltpu.sync_copy(idx_hbm.at[wid, j], idx_v)
            pltpu.sync_copy(r_hbm.at[pl.ds(wid * per_w + j * chunk, chunk)],
                            rows_v)
            pltpu.async_copy(rows_v, out_ref.at[idx_v], sem).wait()
    return s


def _p4_body(qs_ref, wq_ref, wk_ref, bq_ref, a_ref):
    qsp = lax.dot_general(qs_ref[0, 0], wq_ref[...], (((1,), (1,)), ((), ())),
                          preferred_element_type=jnp.float32)
    qsp = qsp + bq_ref[...][None, :]
    a_ref[0, 0] = (jnp.dot(qsp, wk_ref[...],
                           preferred_element_type=jnp.float32)
                   * SCALE).astype(jnp.bfloat16)


def _p4(qsel, Wq, Wk, bq):
    return pl.pallas_call(
        _p4_body,
        grid=(B, H),
        in_specs=[
            pl.BlockSpec((1, 1, UP, D), lambda b, h: (b, h, 0, 0)),
            pl.BlockSpec((Dh, D), lambda b, h: (h, 0)),
            pl.BlockSpec((Dh, D), lambda b, h: (h, 0)),
            pl.BlockSpec((Dh,), lambda b, h: (h,)),
        ],
        out_specs=pl.BlockSpec((1, 1, UP, D), lambda b, h: (b, h, 0, 0)),
        out_shape=jax.ShapeDtypeStruct((B, H, UP, D), jnp.bfloat16),
    )(qsel, Wq, Wk, bq)


def _p5_body(a_ref, c_ref, k_ref, v_ref, p_ref, vs_ref,
             acc_ref, m_ref, l_ref, vsum_ref):
    j = pl.program_id(1)

    @pl.when(j == 0)
    def _():
        acc_ref[...] = jnp.zeros_like(acc_ref)
        m_ref[...] = jnp.full_like(m_ref, NEG)
        l_ref[...] = jnp.zeros_like(l_ref)
        vsum_ref[...] = jnp.zeros_like(vsum_ref)

    s = lax.dot_general(a_ref[0], k_ref[0].astype(jnp.bfloat16),
                        (((1,), (1,)), ((), ())),
                        preferred_element_type=jnp.float32)
    s = s + c_ref[0]
    mcur = jnp.max(s, axis=1, keepdims=True)
    mnew = jnp.maximum(m_ref[...], mcur)
    alpha = jnp.exp(m_ref[...] - mnew)
    p = jnp.exp(s - mnew)
    l_ref[...] = l_ref[...] * alpha + jnp.sum(p, axis=1, keepdims=True)
    acc_ref[...] = acc_ref[...] * alpha + jnp.dot(
        p.astype(jnp.bfloat16), v_ref[0].astype(jnp.bfloat16),
        preferred_element_type=jnp.float32)
    m_ref[...] = mnew
    vsum_ref[...] = vsum_ref[...] + jnp.sum(v_ref[0], axis=0, keepdims=True)

    @pl.when(j == NKB - 1)
    def _():
        p_ref[0] = acc_ref[...] / l_ref[...]
        vs_ref[0] = vsum_ref[...]


def _p5(Acat, c3, k, v):
    return pl.pallas_call(
        _p5_body,
        grid=(B, NKB),
        in_specs=[
            pl.BlockSpec((1, NP, D), lambda b, j: (b, 0, 0)),
            pl.BlockSpec((1, NP, 1), lambda b, j: (b, 0, 0)),
            pl.BlockSpec((1, KBLK, D), lambda b, j: (b, j, 0)),
            pl.BlockSpec((1, KBLK, D), lambda b, j: (b, j, 0)),
        ],
        out_specs=[
            pl.BlockSpec((1, NP, D), lambda b, j: (b, 0, 0)),
            pl.BlockSpec((1, 1, D), lambda b, j: (b, 0, 0)),
        ],
        out_shape=[
            jax.ShapeDtypeStruct((B, NP, D), jnp.float32),
            jax.ShapeDtypeStruct((B, 1, D), jnp.float32),
        ],
        scratch_shapes=[
            pltpu.VMEM((NP, D), jnp.float32),
            pltpu.VMEM((NP, 1), jnp.float32),
            pltpu.VMEM((NP, 1), jnp.float32),
            pltpu.VMEM((1, D), jnp.float32),
        ],
    )(Acat, c3, k, v)


def _p6_body(p_ref, vs_ref, wv_ref, wo_ref, bv_ref, bo_ref, c_ref, base_ref):
    ao = lax.dot_general(p_ref[0, 0], wv_ref[...], (((1,), (1,)), ((), ())),
                         preferred_element_type=jnp.float32)
    vm = lax.dot_general(vs_ref[0] * (1.0 / L), wv_ref[...],
                         (((1,), (1,)), ((), ())),
                         preferred_element_type=jnp.float32)
    delta = ao - vm
    cfull = lax.dot_general(delta, wo_ref[...], (((1,), (1,)), ((), ())),
                            preferred_element_type=jnp.float32)
    zmask = lax.broadcasted_iota(jnp.int32, (UP, 1), 0) < U
    c_ref[0, 0] = jnp.where(zmask, cfull, 0.0)
    bp = lax.dot_general(vm + bv_ref[...][None, :], wo_ref[...],
                         (((1,), (1,)), ((), ())),
                         preferred_element_type=jnp.float32)
    h = pl.program_id(1)

    @pl.when(h == 0)
    def _():
        base_ref[0] = bp + bo_ref[...][None, :]

    @pl.when(h > 0)
    def _():
        base_ref[0] = base_ref[0] + bp


def _p6(P, vs, Wv, Wo, bv, bo):
    return pl.pallas_call(
        _p6_body,
        grid=(B, H),
        in_specs=[
            pl.BlockSpec((1, 1, UP, D), lambda b, h: (b, h, 0, 0)),
            pl.BlockSpec((1, 1, D), lambda b, h: (b, 0, 0)),
            pl.BlockSpec((Dh, D), lambda b, h: (h, 0)),
            pl.BlockSpec((D, Dh), lambda b, h: (0, h)),
            pl.BlockSpec((Dh,), lambda b, h: (h,)),
            pl.BlockSpec((D,), lambda b, h: (0,)),
        ],
        out_specs=[
            pl.BlockSpec((1, 1, UP, D), lambda b, h: (b, h, 0, 0)),
            pl.BlockSpec((1, 1, D), lambda b, h: (b, 0, 0)),
        ],
        out_shape=[
            jax.ShapeDtypeStruct((B, H, UP, D), jnp.float32),
            jax.ShapeDtypeStruct((B, 1, D), jnp.float32),
        ],
    )(P, vs, Wv, Wo, bv, bo)


def _p7_body(ir_ref, ic_ref, c_ref, base_ref, r_ref):
    eq = (ic_ref[0] == ir_ref[0]).astype(jnp.float32)
    r_ref[0] = jnp.dot(eq, c_ref[0],
                       preferred_element_type=jnp.float32) + base_ref[0]


def _p7(gidx_row, gidx_col, Ccat, base):
    return pl.pallas_call(
        _p7_body,
        grid=(B,),
        in_specs=[
            pl.BlockSpec((1, 1, NP), lambda b: (b, 0, 0)),
            pl.BlockSpec((1, NP, 1), lambda b: (b, 0, 0)),
            pl.BlockSpec((1, NP, D), lambda b: (b, 0, 0)),
            pl.BlockSpec((1, 1, D), lambda b: (b, 0, 0)),
        ],
        out_specs=pl.BlockSpec((1, NP, D), lambda b: (b, 0, 0)),
        out_shape=jax.ShapeDtypeStruct((B, NP, D), jnp.float32),
    )(gidx_row, gidx_col, Ccat, base)


def _p8_body(base_ref, out_ref):
    out_ref[...] = jnp.broadcast_to(base_ref[0], out_ref.shape)


def _p8(base):
    return pl.pallas_call(
        _p8_body,
        grid=(B * L // 512,),
        in_specs=[pl.BlockSpec((1, 1, D), lambda j: (j // (L // 512), 0, 0))],
        out_specs=pl.BlockSpec((512, D), lambda j: (j, 0)),
        out_shape=jax.ShapeDtypeStruct((B * L, D), jnp.float32),
    )(base)


def kernel(q, k, v, Wq, bq, Wk, bk, Wv, bv, Wo, bo):
    f32 = jnp.float32
    q, k, v = q.astype(f32), k.astype(f32), v.astype(f32)

    kidx = jax.random.randint(jax.random.key(42), (U,), 0, L)

    kidxp = jnp.concatenate([kidx, jnp.broadcast_to(kidx[:1], (128 - U,))])
    gk = (jnp.arange(B, dtype=jnp.int32)[:, None] * L
          + kidxp[None, :].astype(jnp.int32))
    ksamp_pad = _sc_gather(B * 128, 16)(
        k.reshape(B * L, D), gk.reshape(NW, (B * 128) // NW // 16, 16))
    ksamp = ksamp_pad.reshape(B, 128, D)[:, :U]

    BDT = _p1a(ksamp, Wk, bk)
    pm = jnp.where(jnp.arange(64)[None, :] < U, 0.0, -jnp.inf)
    pm = jnp.broadcast_to(pm, (H, 64)).reshape(H * 64, 1).astype(jnp.float32)
    M = _p1b(q, Wq.astype(jnp.bfloat16), BDT, bq, pm)
    ti = _p2(M.reshape(B * H, L)).reshape(B, H, 64)[:, :, :U]

    ti48 = jnp.concatenate(
        [ti, jnp.broadcast_to(ti[:, :, :1], (B, H, UP - U))], axis=2)
    gidx = (jnp.arange(B, dtype=jnp.int32)[:, None, None] * L
            + ti48)
    gidx_flat = gidx.reshape(NG)

    qsel = _sc_gather(NG, 16)(
        q.reshape(B * L, D),
        gidx_flat.reshape(NW, NG // NW // 16, 16)).reshape(B, H, UP, D)

    A = _p4(qsel, Wq, Wk, bq).reshape(B, NP, D)
    g2 = jnp.einsum('hm,hmd->hd', bk.reshape(H, Dh), Wq.reshape(H, Dh, D))
    c = (jnp.einsum('bhid,hd->bhi', qsel, g2)
         + jnp.einsum('hm,hm->h', bq.reshape(H, Dh),
                      bk.reshape(H, Dh))[None, :, None]) * SCALE
    c3 = c.reshape(B, NP, 1)
    P, vs = _p5(A, c3, k, v)

    C, base = _p6(P.reshape(B, H, UP, D), vs, Wv, Wo, bv, bo)
    gi2 = gidx.reshape(B, NP)
    R = _p7(gi2.reshape(B, 1, NP), gi2.reshape(B, NP, 1),
            C.reshape(B, NP, D), base)

    out0 = _p8(base)
    oref = jax.new_ref(out0)
    _sc_scatter()(R.reshape(NG, D),
                  gidx_flat.reshape(NW, NG // NW // 16, 16), oref)
    return oref[...].reshape(B, L, D)
